# baseline (speedup 1.0000x reference)
import functools

import jax
import jax.numpy as jnp
from jax import lax
from jax.experimental import pallas as pl
from jax.experimental.pallas import tpu as pltpu
from jax.experimental.pallas import tpu_sc as plsc

N = 10000
E = 320000
IN_FEATS = 128
N_HIDDEN = 64
L = 16
K = 10

NW = 16
NP = 10240
RW = NP // NW
CH = 128
NB = 4
NCHUNK = 160
EP = NW * NCHUNK * CH


def _mlp_body(f_ref, w1_ref, b1_ref, w2_ref, b2_ref, o_ref):
    h = jnp.dot(f_ref[...], w1_ref[...], preferred_element_type=jnp.float32)
    h = jnp.maximum(h + b1_ref[...], 0.0)
    o = jnp.dot(h, w2_ref[...], preferred_element_type=jnp.float32)
    o_ref[...] = o + b2_ref[...]


def _mlp(fpad, W1, b1, W2, b2):
    grid = (NP // 1024,)
    return pl.pallas_call(
        _mlp_body,
        grid=grid,
        in_specs=[
            pl.BlockSpec((1024, IN_FEATS), lambda i: (i, 0)),
            pl.BlockSpec((IN_FEATS, N_HIDDEN), lambda i: (0, 0)),
            pl.BlockSpec((1, N_HIDDEN), lambda i: (0, 0)),
            pl.BlockSpec((N_HIDDEN, L), lambda i: (0, 0)),
            pl.BlockSpec((1, L), lambda i: (0, 0)),
        ],
        out_specs=pl.BlockSpec((1024, L), lambda i: (i, 0)),
        out_shape=jax.ShapeDtypeStruct((NP, L), jnp.float32),
    )(fpad, W1, b1, W2, b2)


def _prop(x0, srcp, dstp, normp, tempp, zrow):
    mesh = plsc.VectorSubcoreMesh(
        core_axis_name="c", subcore_axis_name="s", num_cores=1)

    @functools.partial(
        pl.kernel,
        out_type=jax.ShapeDtypeStruct((NP, L), jnp.float32),
        mesh=mesh,
        compiler_params=pltpu.CompilerParams(
            needs_layout_passes=False, use_tc_tiling_on_sc=False),
        scratch_types=[
            pltpu.VMEM_SHARED((NP, L), jnp.float32),
            pltpu.VMEM_SHARED((NP, L), jnp.float32),
            pltpu.HBM((NP, L), jnp.float32),
            pltpu.HBM((NP, L), jnp.float32),
            pltpu.VMEM_SHARED((NP, L), jnp.float32),
            pltpu.VMEM((NCHUNK, CH), jnp.int32),
            pltpu.VMEM((NCHUNK, CH), jnp.int32),
            pltpu.VMEM((NCHUNK * CH,), jnp.float32),
            [pltpu.VMEM((CH, L), jnp.float32) for _ in range(NB)],
            [pltpu.VMEM((CH, L), jnp.float32) for _ in range(NB)],
            pltpu.VMEM((RW, L), jnp.float32),
            pltpu.VMEM((RW, L), jnp.float32),
            pltpu.VMEM((L,), jnp.float32),
            [pltpu.SemaphoreType.DMA for _ in range(NB)],
            [pltpu.SemaphoreType.DMA for _ in range(NB)],
        ],
    )
    def body(x0_h, src_h, dst_h, norm_h, temp_h, z_h, hid_h,
             bufA, bufB, bufAh, bufBh, acc, src_v, dst_v, norm_v, gbuf, sbuf,
             stage_v, hid_v, temp_v, gsem, ssem):
        wid = lax.axis_index("s")
        rows_sl = pl.ds(wid * RW, RW)

        pltpu.sync_copy(src_h.at[wid], src_v)
        pltpu.sync_copy(dst_h.at[wid], dst_v)
        pltpu.sync_copy(norm_h.at[wid], norm_v)
        pltpu.sync_copy(temp_h, temp_v)

        pltpu.sync_copy(x0_h.at[rows_sl], stage_v)
        t0 = plsc.load_gather(temp_v, [jnp.zeros((L,), jnp.int32)])

        @pl.loop(0, RW)
        def initrow(i):
            hid_v[i] = stage_v[i] * t0
        pltpu.sync_copy(z_h, acc.at[rows_sl])
        pltpu.sync_copy(stage_v, bufA.at[rows_sl])
        pltpu.sync_copy(stage_v, bufAh.at[rows_sl])
        plsc.subcore_barrier()

        def hop(kk, xsrc, xsrch, xdst, xdsth):
            for b in range(NB):
                xs = xsrch if b == 0 else xsrc
                pltpu.async_copy(xs.at[src_v.at[b]], gbuf[b], gsem[b])

            @pl.loop(0, NCHUNK // NB)
            def group(i0):
                for b in range(NB):
                    i = i0 * NB + b
                    xs = xsrch if b == 0 else xsrc
                    pltpu.make_async_copy(
                        xs.at[src_v.at[i]], gbuf[b], gsem[b]).wait()

                    @pl.when(i0 > 0)
                    def _():
                        pltpu.make_async_copy(
                            sbuf[b], acc.at[dst_v.at[i]], ssem[b]).wait()
                    base = i * CH

                    @plsc.parallel_loop(0, CH, unroll=16)
                    def edge(j):
                        nb = plsc.load_gather(
                            norm_v, [jnp.full((L,), base + j, jnp.int32)])
                        sbuf[b][j] = gbuf[b][j] * nb

                    @pl.when(i0 < NCHUNK // NB - 1)
                    def _():
                        pltpu.async_copy(
                            xs.at[src_v.at[i + NB]], gbuf[b], gsem[b])
                    pltpu.async_copy(
                        sbuf[b], acc.at[dst_v.at[i]], ssem[b], add=True)

            for b in range(NB):
                pltpu.make_async_copy(
                    sbuf[b], acc.at[dst_v.at[NCHUNK - NB + b]], ssem[b]).wait()
            plsc.subcore_barrier()

            pltpu.sync_copy(acc.at[rows_sl], stage_v)
            tk = plsc.load_gather(
                temp_v, [jnp.full((L,), kk + 1, jnp.int32)])

            @pl.loop(0, RW, unroll=16)
            def acrow(i):
                hid_v[i] = hid_v[i] + stage_v[i] * tk
            pltpu.sync_copy(stage_v, xdst.at[rows_sl])
            pltpu.sync_copy(stage_v, xdsth.at[rows_sl])
            pltpu.sync_copy(z_h, acc.at[rows_sl])
            plsc.subcore_barrier()

        @pl.loop(0, K // 2)
        def hoppair(i0):
            hop(2 * i0, bufA, bufAh, bufB, bufBh)
            hop(2 * i0 + 1, bufB, bufBh, bufA, bufAh)

        pltpu.sync_copy(hid_v, hid_h.at[rows_sl])

    return body(x0, srcp, dstp, normp, tempp, zrow)


def kernel(feature, edge_index, norm_A, W1, b1, W2, b2, temp):
    fpad = jnp.pad(feature, ((0, NP - N), (0, 0)))
    x0 = _mlp(fpad, W1, b1.reshape(1, -1), W2, b2.reshape(1, -1))

    src = jnp.pad(edge_index[0], (0, EP - E)).reshape(NW, NCHUNK, CH)
    dst = jnp.pad(edge_index[1], (0, EP - E)).reshape(NW, NCHUNK, CH)
    nrm = jnp.pad(norm_A, (0, EP - E)).reshape(NW, NCHUNK * CH)
    tpad = jnp.pad(temp, (0, L - (K + 1)))
    zrow = jnp.zeros((RW, L), jnp.float32)

    hid = _prop(x0, src, dst, nrm, tpad, zrow)
    return hid[:N]


# speedup vs baseline: 1.0903x; 1.0903x over previous
import functools

import jax
import jax.numpy as jnp
from jax import lax
from jax.experimental import pallas as pl
from jax.experimental.pallas import tpu as pltpu
from jax.experimental.pallas import tpu_sc as plsc

N = 10000
E = 320000
IN_FEATS = 128
N_HIDDEN = 64
L = 16
K = 10

NW = 16
NP = 10240
RW = NP // NW
CH = 128
GC = 256
NB = 4
NG = 80
EP = NW * NG * GC


def _mlp_body(f_ref, w1_ref, b1_ref, w2_ref, b2_ref, o_ref):
    h = jnp.dot(f_ref[...], w1_ref[...], preferred_element_type=jnp.float32)
    h = jnp.maximum(h + b1_ref[...], 0.0)
    o = jnp.dot(h, w2_ref[...], preferred_element_type=jnp.float32)
    o_ref[...] = o + b2_ref[...]


def _mlp(fpad, W1, b1, W2, b2):
    grid = (NP // 1024,)
    return pl.pallas_call(
        _mlp_body,
        grid=grid,
        in_specs=[
            pl.BlockSpec((1024, IN_FEATS), lambda i: (i, 0)),
            pl.BlockSpec((IN_FEATS, N_HIDDEN), lambda i: (0, 0)),
            pl.BlockSpec((1, N_HIDDEN), lambda i: (0, 0)),
            pl.BlockSpec((N_HIDDEN, L), lambda i: (0, 0)),
            pl.BlockSpec((1, L), lambda i: (0, 0)),
        ],
        out_specs=pl.BlockSpec((1024, L), lambda i: (i, 0)),
        out_shape=jax.ShapeDtypeStruct((NP, L), jnp.float32),
    )(fpad, W1, b1, W2, b2)


def _prop(x0, srcp, dstp, normp, tempp, zrow):
    mesh = plsc.VectorSubcoreMesh(
        core_axis_name="c", subcore_axis_name="s", num_cores=1)

    @functools.partial(
        pl.kernel,
        out_type=jax.ShapeDtypeStruct((NP, L), jnp.float32),
        mesh=mesh,
        compiler_params=pltpu.CompilerParams(
            needs_layout_passes=False, use_tc_tiling_on_sc=False),
        scratch_types=[
            pltpu.VMEM_SHARED((NP, L), jnp.float32),
            pltpu.VMEM_SHARED((NP, L), jnp.float32),
            pltpu.VMEM_SHARED((NP, L), jnp.float32),
            pltpu.VMEM((NG, GC), jnp.int32),
            pltpu.VMEM((NG, GC), jnp.int32),
            [pltpu.VMEM((GC,), jnp.float32) for _ in range(NB)],
            [pltpu.VMEM((GC, L), jnp.float32) for _ in range(NB)],
            [pltpu.VMEM((GC, L), jnp.float32) for _ in range(NB)],
            pltpu.VMEM((RW, L), jnp.float32),
            pltpu.VMEM((RW, L), jnp.float32),
            pltpu.VMEM((L,), jnp.float32),
            [pltpu.SemaphoreType.DMA for _ in range(NB)],
            [pltpu.SemaphoreType.DMA for _ in range(NB)],
            [pltpu.SemaphoreType.DMA for _ in range(NB)],
        ],
    )
    def body(x0_h, src_h, dst_h, norm_h, temp_h, z_h, hid_h,
             bufA, bufB, acc, src_v, dst_v, normb, gbuf, sbuf,
             stage_v, hid_v, temp_v, gsem, ssem, nsem):
        wid = lax.axis_index("s")
        rows_sl = pl.ds(wid * RW, RW)

        pltpu.sync_copy(src_h.at[wid], src_v)
        pltpu.sync_copy(dst_h.at[wid], dst_v)
        pltpu.sync_copy(temp_h, temp_v)
        nrm = norm_h.at[wid]

        pltpu.sync_copy(x0_h.at[rows_sl], stage_v)
        t0 = plsc.load_gather(temp_v, [jnp.zeros((L,), jnp.int32)])

        @pl.loop(0, RW)
        def initrow(i):
            hid_v[i] = stage_v[i] * t0
        pltpu.sync_copy(z_h, acc.at[rows_sl])
        pltpu.sync_copy(stage_v, bufA.at[rows_sl])
        plsc.subcore_barrier()

        def hop(kk, xsrc, xdst):
            for b in range(NB):
                pltpu.async_copy(xsrc.at[src_v.at[b]], gbuf[b], gsem[b])
                pltpu.async_copy(nrm.at[b], normb[b], nsem[b])

            @pl.loop(0, NG // NB)
            def group(i0):
                for b in range(NB):
                    g = i0 * NB + b
                    pltpu.make_async_copy(
                        xsrc.at[src_v.at[g]], gbuf[b], gsem[b]).wait()
                    pltpu.make_async_copy(nrm.at[g], normb[b], nsem[b]).wait()

                    @pl.when(i0 > 0)
                    def _():
                        pltpu.make_async_copy(
                            sbuf[b], acc.at[dst_v.at[g]], ssem[b]).wait()
                    @plsc.parallel_loop(0, GC, unroll=16)
                    def edge(j):
                        nb = plsc.load_gather(
                            normb[b], [jnp.full((L,), j, jnp.int32)])
                        sbuf[b][j] = gbuf[b][j] * nb

                    @pl.when(i0 < NG // NB - 1)
                    def _():
                        pltpu.async_copy(
                            xsrc.at[src_v.at[g + NB]], gbuf[b], gsem[b])
                        pltpu.async_copy(nrm.at[g + NB], normb[b], nsem[b])
                    pltpu.async_copy(
                        sbuf[b], acc.at[dst_v.at[g]], ssem[b], add=True)

            for b in range(NB):
                pltpu.make_async_copy(
                    sbuf[b], acc.at[dst_v.at[NG - NB + b]], ssem[b]).wait()
            plsc.subcore_barrier()

            pltpu.sync_copy(acc.at[rows_sl], stage_v)
            tk = plsc.load_gather(
                temp_v, [jnp.full((L,), kk + 1, jnp.int32)])

            @pl.loop(0, RW, unroll=16)
            def acrow(i):
                hid_v[i] = hid_v[i] + stage_v[i] * tk
            pltpu.sync_copy(stage_v, xdst.at[rows_sl])
            pltpu.sync_copy(z_h, acc.at[rows_sl])
            plsc.subcore_barrier()

        @pl.loop(0, K // 2)
        def hoppair(i0):
            hop(2 * i0, bufA, bufB)
            hop(2 * i0 + 1, bufB, bufA)

        pltpu.sync_copy(hid_v, hid_h.at[rows_sl])

    return body(x0, srcp, dstp, normp, tempp, zrow)


def kernel(feature, edge_index, norm_A, W1, b1, W2, b2, temp):
    fpad = jnp.pad(feature, ((0, NP - N), (0, 0)))
    x0 = _mlp(fpad, W1, b1.reshape(1, -1), W2, b2.reshape(1, -1))

    src = jnp.pad(edge_index[0], (0, EP - E)).reshape(NW, NG, GC)
    dst = jnp.pad(edge_index[1], (0, EP - E)).reshape(NW, NG, GC)
    nrm = jnp.pad(norm_A, (0, EP - E)).reshape(NW, NG, GC)
    tpad = jnp.pad(temp, (0, L - (K + 1)))
    zrow = jnp.zeros((RW, L), jnp.float32)

    hid = _prop(x0, src, dst, nrm, tpad, zrow)
    return hid[:N]


# speedup vs baseline: 1.1759x; 1.0785x over previous
import functools

import jax
import jax.numpy as jnp
from jax import lax
from jax.experimental import pallas as pl
from jax.experimental.pallas import tpu as pltpu
from jax.experimental.pallas import tpu_sc as plsc

N = 10000
E = 320000
IN_FEATS = 128
N_HIDDEN = 64
L = 16
K = 10

NW = 16
NP = 10240
RW = NP // NW
CH = 128
NB = 4
NCHUNK = 160
EP = NW * NCHUNK * CH


def _mlp_body(f_ref, w1_ref, b1_ref, w2_ref, b2_ref, o_ref):
    h = jnp.dot(f_ref[...], w1_ref[...], preferred_element_type=jnp.float32)
    h = jnp.maximum(h + b1_ref[...], 0.0)
    o = jnp.dot(h, w2_ref[...], preferred_element_type=jnp.float32)
    o_ref[...] = o + b2_ref[...]


def _mlp(fpad, W1, b1, W2, b2):
    grid = (NP // 1024,)
    return pl.pallas_call(
        _mlp_body,
        grid=grid,
        in_specs=[
            pl.BlockSpec((1024, IN_FEATS), lambda i: (i, 0)),
            pl.BlockSpec((IN_FEATS, N_HIDDEN), lambda i: (0, 0)),
            pl.BlockSpec((1, N_HIDDEN), lambda i: (0, 0)),
            pl.BlockSpec((N_HIDDEN, L), lambda i: (0, 0)),
            pl.BlockSpec((1, L), lambda i: (0, 0)),
        ],
        out_specs=pl.BlockSpec((1024, L), lambda i: (i, 0)),
        out_shape=jax.ShapeDtypeStruct((NP, L), jnp.float32),
    )(fpad, W1, b1, W2, b2)


def _prop(x0, srcp, dstp, normp, tempp, zrow):
    mesh = plsc.VectorSubcoreMesh(
        core_axis_name="c", subcore_axis_name="s", num_cores=1)

    @functools.partial(
        pl.kernel,
        out_type=jax.ShapeDtypeStruct((NP, L), jnp.float32),
        mesh=mesh,
        compiler_params=pltpu.CompilerParams(
            needs_layout_passes=False, use_tc_tiling_on_sc=False),
        scratch_types=[
            pltpu.VMEM_SHARED((NP, L), jnp.float32),
            pltpu.VMEM_SHARED((NP, L), jnp.float32),
            pltpu.VMEM_SHARED((NP, L), jnp.float32),
            pltpu.VMEM((NCHUNK, CH), jnp.int32),
            pltpu.VMEM((NCHUNK, CH), jnp.int32),
            pltpu.VMEM((NCHUNK * CH,), jnp.float32),
            [pltpu.VMEM((CH, L), jnp.float32) for _ in range(NB)],
            [pltpu.VMEM((CH, L), jnp.float32) for _ in range(NB)],
            pltpu.VMEM((RW, L), jnp.float32),
            pltpu.VMEM((RW, L), jnp.float32),
            pltpu.VMEM((L,), jnp.float32),
            [pltpu.SemaphoreType.DMA for _ in range(NB)],
            [pltpu.SemaphoreType.DMA for _ in range(NB)],
            pltpu.SemaphoreType.DMA,
            pltpu.SemaphoreType.DMA,
        ],
    )
    def body(x0_h, src_h, dst_h, norm_h, temp_h, z_h, hid_h,
             bufA, bufB, acc, src_v, dst_v, norm_v, gbuf, sbuf,
             stage_v, hid_v, temp_v, gsem, ssem, esema, esemb):
        wid = lax.axis_index("s")
        rows_sl = pl.ds(wid * RW, RW)

        pltpu.sync_copy(src_h.at[wid], src_v)
        pltpu.sync_copy(dst_h.at[wid], dst_v)
        pltpu.sync_copy(norm_h.at[wid], norm_v)
        pltpu.sync_copy(temp_h, temp_v)

        pltpu.sync_copy(x0_h.at[rows_sl], stage_v)
        t0 = plsc.load_gather(temp_v, [jnp.zeros((L,), jnp.int32)])

        @pl.loop(0, RW)
        def initrow(i):
            hid_v[i] = stage_v[i] * t0
        pltpu.sync_copy(z_h, acc.at[rows_sl])
        pltpu.sync_copy(stage_v, bufA.at[rows_sl])
        plsc.subcore_barrier()

        def hop(kk, xsrc, xdst):
            for b in range(NB):
                pltpu.async_copy(xsrc.at[src_v.at[b]], gbuf[b], gsem[b])

            @pl.loop(0, NCHUNK // NB)
            def group(i0):
                for b in range(NB):
                    i = i0 * NB + b
                    pltpu.make_async_copy(
                        xsrc.at[src_v.at[i]], gbuf[b], gsem[b]).wait()

                    @pl.when(i0 > 0)
                    def _():
                        pltpu.make_async_copy(
                            sbuf[b], acc.at[dst_v.at[i]], ssem[b]).wait()
                    base = i * CH

                    @plsc.parallel_loop(0, CH, unroll=16)
                    def edge(j):
                        nb = plsc.load_gather(
                            norm_v, [jnp.full((L,), base + j, jnp.int32)])
                        sbuf[b][j] = gbuf[b][j] * nb

                    @pl.when(i0 < NCHUNK // NB - 1)
                    def _():
                        pltpu.async_copy(
                            xsrc.at[src_v.at[i + NB]], gbuf[b], gsem[b])
                    pltpu.async_copy(
                        sbuf[b], acc.at[dst_v.at[i]], ssem[b], add=True)

            for b in range(NB):
                pltpu.make_async_copy(
                    sbuf[b], acc.at[dst_v.at[NCHUNK - NB + b]], ssem[b]).wait()
            plsc.subcore_barrier()

            pltpu.sync_copy(acc.at[rows_sl], stage_v)
            tk = plsc.load_gather(
                temp_v, [jnp.full((L,), kk + 1, jnp.int32)])

            pltpu.async_copy(stage_v, xdst.at[rows_sl], esema)
            pltpu.async_copy(z_h, acc.at[rows_sl], esemb)

            @pl.loop(0, RW, unroll=16)
            def acrow(i):
                hid_v[i] = hid_v[i] + stage_v[i] * tk
            pltpu.make_async_copy(stage_v, xdst.at[rows_sl], esema).wait()
            pltpu.make_async_copy(z_h, acc.at[rows_sl], esemb).wait()
            plsc.subcore_barrier()

        @pl.loop(0, K // 2)
        def hoppair(i0):
            hop(2 * i0, bufA, bufB)
            hop(2 * i0 + 1, bufB, bufA)

        pltpu.sync_copy(hid_v, hid_h.at[rows_sl])

    return body(x0, srcp, dstp, normp, tempp, zrow)


def kernel(feature, edge_index, norm_A, W1, b1, W2, b2, temp):
    fpad = jnp.pad(feature, ((0, NP - N), (0, 0)))
    x0 = _mlp(fpad, W1, b1.reshape(1, -1), W2, b2.reshape(1, -1))

    src = jnp.pad(edge_index[0], (0, EP - E)).reshape(NW, NCHUNK, CH)
    dst = jnp.pad(edge_index[1], (0, EP - E)).reshape(NW, NCHUNK, CH)
    nrm = jnp.pad(norm_A, (0, EP - E)).reshape(NW, NCHUNK * CH)
    tpad = jnp.pad(temp, (0, L - (K + 1)))
    zrow = jnp.zeros((RW, L), jnp.float32)

    hid = _prop(x0, src, dst, nrm, tpad, zrow)
    return hid[:N]
